# baseline (speedup 1.0000x reference)
import functools

import jax
import jax.numpy as jnp
import numpy as np
from jax import lax
from jax.experimental import pallas as pl
from jax.experimental.pallas import tpu as pltpu
from jax.experimental.pallas import tpu_sc as plsc

def _bf2(v):
    a = lax.bitcast_convert_type(lax.shift_left(v, 16), jnp.float32)
    b = lax.bitcast_convert_type(lax.bitwise_and(v, jnp.int32(-65536)),
                                 jnp.float32)
    return a, b

N = 10000
D = 256
E = 160000
EPS = 1e-7
MIN_NORM = 1e-15

NC, NS, L = 2, 16, 16
NW = NC * NS
E_PAD = 163840
PACKW = 512

C1 = 64
EPT1 = E_PAD // NW
NCH1 = EPT1 // C1

C2 = 128
EPT3 = E_PAD // NS
NCH3 = EPT3 // C2
N_PAD = 10240
NPT = N_PAD // NS

BN = 1000
BE = 2048

_mesh = plsc.VectorSubcoreMesh(core_axis_name="c", subcore_axis_name="s")


def _prep_body(x_ref, w1a_ref, w1b_ref,
               packR_ref, packC_ref, xtL_ref, xtR_ref, xtan_ref):
    x = x_ref[...]
    x2 = jnp.sum(x * x, axis=1, keepdims=True)
    n = jnp.sqrt(jnp.maximum(x2, MIN_NORM))
    u = jnp.clip(n, -1.0 + EPS, 1.0 - EPS)
    art = 0.5 * (jnp.log1p(u) - jnp.log1p(-u))
    t = art / n
    xt = x * t
    pa = jnp.dot(x, w1a_ref[...], preferred_element_type=jnp.float32)
    pb = jnp.dot(x, w1b_ref[...], preferred_element_type=jnp.float32)
    packR_ref[...] = jnp.concatenate([xt, pa], axis=1).astype(jnp.bfloat16)
    packC_ref[...] = jnp.concatenate([xt, pb], axis=1).astype(jnp.bfloat16)
    xtL_ref[...] = xt[:, :128]
    xtR_ref[...] = xt[:, 128:]
    xtan_ref[...] = xt


_prep_call = pl.pallas_call(
    _prep_body,
    grid=(N // BN,),
    in_specs=[
        pl.BlockSpec((BN, D), lambda n: (n, 0)),
        pl.BlockSpec((D, D), lambda n: (0, 0)),
        pl.BlockSpec((D, D), lambda n: (0, 0)),
    ],
    out_specs=[
        pl.BlockSpec((BN, PACKW), lambda n: (n, 0)),
        pl.BlockSpec((BN, PACKW), lambda n: (n, 0)),
        pl.BlockSpec((BN, 128), lambda n: (n, 0)),
        pl.BlockSpec((BN, 128), lambda n: (n, 0)),
        pl.BlockSpec((BN, D), lambda n: (n, 0)),
    ],
    out_shape=[
        jax.ShapeDtypeStruct((N, PACKW), jnp.bfloat16),
        jax.ShapeDtypeStruct((N, PACKW), jnp.bfloat16),
        jax.ShapeDtypeStruct((N, 128), jnp.float32),
        jax.ShapeDtypeStruct((N, 128), jnp.float32),
        jax.ShapeDtypeStruct((N, D), jnp.float32),
    ],
)


@functools.partial(
    pl.kernel,
    out_type=[
        jax.ShapeDtypeStruct((E_PAD, D), jnp.float32),
        jax.ShapeDtypeStruct((E_PAD, 48), jnp.float32),
    ],
    mesh=_mesh,
    scratch_types=[
        pltpu.VMEM((C1,), jnp.int32),
        pltpu.VMEM((C1,), jnp.int32),
        pltpu.VMEM((C1, PACKW // 2), jnp.int32),
        pltpu.VMEM((C1, PACKW // 2), jnp.int32),
        pltpu.VMEM((C1, D), jnp.float32),
        pltpu.VMEM((C1, 48), jnp.float32),
        pltpu.SemaphoreType.DMA,
        pltpu.SemaphoreType.DMA,
    ],
)
def _sc_edge(packR, packC, rowi, coli, pre_out, scal_out,
             idxr, idxc, bufR, bufC, preB, scalB, semR, semC):
    wid = lax.axis_index("s") * NC + lax.axis_index("c")
    tbase = wid * EPT1

    def chunk(g, carry):
        base = tbase + g * C1
        pltpu.sync_copy(rowi.at[pl.ds(base, C1)], idxr)
        pltpu.sync_copy(coli.at[pl.ds(base, C1)], idxc)
        cp_r = pltpu.async_copy(packR.at[idxr], bufR, semR)
        cp_c = pltpu.async_copy(packC.at[idxc], bufC, semC)
        cp_r.wait()
        cp_c.wait()

        def edge(i):
            acc = jnp.zeros((L,), jnp.float32)
            acr = jnp.zeros((L,), jnp.float32)
            acc_c = jnp.zeros((L,), jnp.float32)
            for j in range(D // (2 * L)):
                ra, rb = _bf2(bufR[i, pl.ds(L * j, L)])
                ca, cb = _bf2(bufC[i, pl.ds(L * j, L)])
                acc = acc + ra * ca + rb * cb
                acr = acr + ra * ra + rb * rb
                acc_c = acc_c + ca * ca + cb * cb
            scalB[i, pl.ds(0, L)] = acc
            scalB[i, pl.ds(16, L)] = acr
            scalB[i, pl.ds(32, L)] = acc_c
            for j in range(D // (2 * L)):
                ra, rb = _bf2(bufR[i, pl.ds(D // 2 + L * j, L)])
                ca, cb = _bf2(bufC[i, pl.ds(D // 2 + L * j, L)])
                preB[i, pl.ds(2 * L * j, L)] = ra + ca
                preB[i, pl.ds(2 * L * j + L, L)] = rb + cb

        plsc.parallel_loop(0, C1, unroll=2)(edge)
        pltpu.sync_copy(preB, pre_out.at[pl.ds(base, C1)])
        pltpu.sync_copy(scalB, scal_out.at[pl.ds(base, C1)])
        return carry

    lax.fori_loop(0, NCH1, chunk, 0)


def _edge_body(pre_ref, scal_ref, dd_ref, em_ref,
               w1d_ref, b1_ref, w2_ref, b2_ref, srep_ref):
    sc = scal_ref[...]
    dot = jnp.sum(sc[:, 0:16], axis=1, keepdims=True)
    art_r = jnp.sqrt(jnp.maximum(jnp.sum(sc[:, 16:32], axis=1, keepdims=True),
                                 MIN_NORM))
    art_c = jnp.sqrt(jnp.maximum(jnp.sum(sc[:, 32:48], axis=1, keepdims=True),
                                 MIN_NORM))
    nr = jnp.tanh(art_r)
    nc = jnp.tanh(art_c)
    x2r = nr * nr
    y2 = nc * nc
    tr = art_r / nr
    tc_ = art_c / nc
    xy = dot / (tr * tc_)
    a = 1.0 - 2.0 * xy + y2
    b = 1.0 - x2r
    den = jnp.maximum(1.0 - 2.0 * xy + x2r * y2, MIN_NORM)
    nsq = (a * a * x2r - 2.0 * a * b * xy + b * b * y2) / (den * den)
    nn = jnp.sqrt(jnp.maximum(nsq, MIN_NORM))
    u = jnp.clip(nn, -1.0 + EPS, 1.0 - EPS)
    dist = jnp.log1p(u) - jnp.log1p(-u)
    z = (pre_ref[...] + dist * w1d_ref[0:1, :]
         + dd_ref[...] * w1d_ref[1:2, :] + b1_ref[...])
    h = z / (1.0 + jnp.exp(-z))
    s = jnp.dot(h, w2_ref[...], preferred_element_type=jnp.float32) + b2_ref[...]
    score = em_ref[...] / (1.0 + jnp.exp(-s))
    srep_ref[...] = jnp.broadcast_to(score, (score.shape[0], 16))


_edge_call = pl.pallas_call(
    _edge_body,
    grid=(E_PAD // BE,),
    in_specs=[
        pl.BlockSpec((BE, D), lambda n: (n, 0)),
        pl.BlockSpec((BE, 48), lambda n: (n, 0)),
        pl.BlockSpec((BE, 1), lambda n: (n, 0)),
        pl.BlockSpec((BE, 1), lambda n: (n, 0)),
        pl.BlockSpec((2, D), lambda n: (0, 0)),
        pl.BlockSpec((1, D), lambda n: (0, 0)),
        pl.BlockSpec((D, 1), lambda n: (0, 0)),
        pl.BlockSpec((1, 1), lambda n: (0, 0)),
    ],
    out_specs=pl.BlockSpec((BE, 16), lambda n: (n, 0)),
    out_shape=jax.ShapeDtypeStruct((E_PAD, 16), jnp.float32),
)


@functools.partial(
    pl.kernel,
    out_type=[
        jax.ShapeDtypeStruct((N_PAD, 128), jnp.float32),
        jax.ShapeDtypeStruct((N_PAD, 128), jnp.float32),
    ],
    mesh=_mesh,
    scratch_types=[
        pltpu.VMEM((C2,), jnp.int32),
        pltpu.VMEM((C2,), jnp.int32),
        pltpu.VMEM((C2, 128), jnp.float32),
        pltpu.VMEM((C2, 16), jnp.float32),
        pltpu.VMEM_SHARED((N_PAD, 128), jnp.float32),
        pltpu.SemaphoreType.DMA,
    ],
)
def _sc_scatter(xt2, rowi, coli, srep, zrows, aggL_out, aggR_out,
                idxc, idxr, vbuf, sbuf, acc, sem):
    cid = lax.axis_index("c")
    sid = lax.axis_index("s")
    pltpu.sync_copy(zrows, acc.at[pl.ds(sid * NPT, NPT)])
    plsc.subcore_barrier()
    off = cid * N

    def chunk(g, carry):
        base = sid * EPT3 + g * C2
        pltpu.sync_copy(coli.at[pl.ds(base, C2)], idxc)
        pltpu.sync_copy(rowi.at[pl.ds(base, C2)], idxr)
        pltpu.sync_copy(srep.at[pl.ds(base, C2)], sbuf)
        for q in range(C2 // L):
            idxc[pl.ds(q * L, L)] = idxc[pl.ds(q * L, L)] + off
        pltpu.async_copy(xt2.at[idxc], vbuf, sem).wait()

        def edge(i):
            sv = sbuf[i, :]
            for j in range(128 // L):
                vbuf[i, pl.ds(L * j, L)] = vbuf[i, pl.ds(L * j, L)] * sv

        plsc.parallel_loop(0, C2, unroll=2)(edge)
        pltpu.sync_copy(vbuf, acc.at[idxr], add=True)
        return carry

    lax.fori_loop(0, NCH3, chunk, 0)
    plsc.subcore_barrier()

    @pl.when(cid == 0)
    def _():
        pltpu.sync_copy(acc.at[pl.ds(sid * NPT, NPT)],
                        aggL_out.at[pl.ds(sid * NPT, NPT)])

    @pl.when(cid == 1)
    def _():
        pltpu.sync_copy(acc.at[pl.ds(sid * NPT, NPT)],
                        aggR_out.at[pl.ds(sid * NPT, NPT)])


def _post_body(aL_ref, aR_ref, xt_ref, wm1_ref, bm1_ref, wm2_ref, bm2_ref,
               out_ref):
    agg = jnp.concatenate([aL_ref[...], aR_ref[...]], axis=1) * 0.01
    z = jnp.dot(agg, wm1_ref[...], preferred_element_type=jnp.float32) + bm1_ref[...]
    h = z / (1.0 + jnp.exp(-z))
    u = (jnp.dot(h, wm2_ref[...], preferred_element_type=jnp.float32)
         + bm2_ref[...] + xt_ref[...])
    nsq = jnp.sum(u * u, axis=1, keepdims=True)
    n = jnp.sqrt(jnp.maximum(nsq, MIN_NORM))
    out_ref[...] = jnp.tanh(n) * u / n


_post_call = pl.pallas_call(
    _post_body,
    grid=(N // BN,),
    in_specs=[
        pl.BlockSpec((BN, 128), lambda n: (n, 0)),
        pl.BlockSpec((BN, 128), lambda n: (n, 0)),
        pl.BlockSpec((BN, D), lambda n: (n, 0)),
        pl.BlockSpec((D, D), lambda n: (0, 0)),
        pl.BlockSpec((1, D), lambda n: (0, 0)),
        pl.BlockSpec((D, D), lambda n: (0, 0)),
        pl.BlockSpec((1, D), lambda n: (0, 0)),
    ],
    out_specs=pl.BlockSpec((BN, D), lambda n: (n, 0)),
    out_shape=jax.ShapeDtypeStruct((N, D), jnp.float32),
)


def _unpack_perm(width):
    p = np.zeros(width, np.int32)
    for s in range(width):
        g, r = s // 32, s % 32
        p[s] = 32 * g + (2 * r if r < 16 else 2 * (r - 16) + 1)
    return p


_PERM_D = _unpack_perm(D)
_PERM_AGG = np.concatenate([_unpack_perm(128), 128 + _unpack_perm(128)])


def kernel(x, distances, edges, node_mask, edge_mask,
           W_att1, b_att1, W_att2, b_att2, W_m1, b_m1, W_m2, b_m2):
    row = edges[0]
    col = edges[1]
    rowp = jnp.pad(row, (0, E_PAD - E))
    colp = jnp.pad(col, (0, E_PAD - E))
    dd = jnp.pad(distances, ((0, E_PAD - E), (0, 0)))
    em = jnp.pad(edge_mask, ((0, E_PAD - E), (0, 0)))
    w1a = W_att1[:D]
    w1b = W_att1[D:2 * D]
    w1d = W_att1[2 * D:]
    b1 = b_att1.reshape(1, D)
    b2 = b_att2.reshape(1, 1)
    bm1 = b_m1.reshape(1, D)
    bm2 = b_m2.reshape(1, D)
    zrows = jnp.zeros((NPT, 128), jnp.float32)

    packR, packC, xtL, xtR, x_tan = _prep_call(x, w1a, w1b)
    packR32 = lax.bitcast_convert_type(packR.reshape(N, PACKW // 2, 2),
                                       jnp.int32)
    packC32 = lax.bitcast_convert_type(packC.reshape(N, PACKW // 2, 2),
                                       jnp.int32)
    xt2 = jnp.concatenate([xtL, xtR], axis=0)
    pre_e, scal_e = _sc_edge(packR32, packC32, rowp, colp)
    srep = _edge_call(pre_e, scal_e, dd, em, w1d[:, _PERM_D], b1[:, _PERM_D],
                      W_att2[_PERM_D], b2)
    aggL, aggR = _sc_scatter(xt2, rowp, colp, srep, zrows)
    out = _post_call(aggL, aggR, x_tan, W_m1, bm1, W_m2, bm2)
    return out


# speedup vs baseline: 1.1828x; 1.1828x over previous
import functools

import jax
import jax.numpy as jnp
import numpy as np
from jax import lax
from jax.experimental import pallas as pl
from jax.experimental.pallas import tpu as pltpu
from jax.experimental.pallas import tpu_sc as plsc

N = 10000
D = 256
E = 160000
EPS = 1e-7
MIN_NORM = 1e-15

NC, NS, L = 2, 16, 16
NW = NC * NS
E_PAD = 163840
PACKW = 512
PACKW32 = PACKW // 2

C1 = 64
EPT1 = E_PAD // NW
NCH1 = EPT1 // C1
OW = 304

C2 = 64
EPT3 = E_PAD // NS
NCH3 = EPT3 // C2
N_PAD = 10112
NPT = N_PAD // NS

BN = 1000
BE = 2048

_mesh = plsc.VectorSubcoreMesh(core_axis_name="c", subcore_axis_name="s")


def _bf2(v):
    a = lax.bitcast_convert_type(lax.shift_left(v, 16), jnp.float32)
    b = lax.bitcast_convert_type(lax.bitwise_and(v, jnp.int32(-65536)),
                                 jnp.float32)
    return a, b


def _unpack_perm(width):
    p = np.zeros(width, np.int32)
    for s in range(width):
        g, r = s // 32, s % 32
        p[s] = 32 * g + (2 * r if r < 16 else 2 * (r - 16) + 1)
    return p


_PERM_D = _unpack_perm(D)


def _prep_body(x_ref, w_ref, pack_ref, xtL_ref, xtR_ref, xtan_ref):
    x = x_ref[...]
    x2 = jnp.sum(x * x, axis=1, keepdims=True)
    n = jnp.sqrt(jnp.maximum(x2, MIN_NORM))
    u = jnp.clip(n, -1.0 + EPS, 1.0 - EPS)
    art = 0.5 * (jnp.log1p(u) - jnp.log1p(-u))
    t = art / n
    xt = x * t
    p = jnp.dot(x, w_ref[0], preferred_element_type=jnp.float32)
    pack_ref[...] = jnp.concatenate([xt, p], axis=1).astype(jnp.bfloat16)
    xtL_ref[...] = xt[:, :128]
    xtR_ref[...] = xt[:, 128:]
    xtan_ref[...] = xt


_prep_call = pl.pallas_call(
    _prep_body,
    grid=(2, N // BN),
    in_specs=[
        pl.BlockSpec((BN, D), lambda p, n: (n, 0)),
        pl.BlockSpec((1, D, D), lambda p, n: (p, 0, 0)),
    ],
    out_specs=[
        pl.BlockSpec((BN, PACKW), lambda p, n: (p * (N // BN) + n, 0)),
        pl.BlockSpec((BN, 128), lambda p, n: (n, 0)),
        pl.BlockSpec((BN, 128), lambda p, n: (n, 0)),
        pl.BlockSpec((BN, D), lambda p, n: (n, 0)),
    ],
    out_shape=[
        jax.ShapeDtypeStruct((2 * N, PACKW), jnp.bfloat16),
        jax.ShapeDtypeStruct((N, 128), jnp.float32),
        jax.ShapeDtypeStruct((N, 128), jnp.float32),
        jax.ShapeDtypeStruct((N, D), jnp.float32),
    ],
)


@functools.partial(
    pl.kernel,
    out_type=jax.ShapeDtypeStruct((E_PAD, OW), jnp.float32),
    mesh=_mesh,
    scratch_types=[
        pltpu.VMEM((2 * C1,), jnp.int32),
        pltpu.VMEM((2 * C1,), jnp.int32),
        pltpu.VMEM((2 * C1, PACKW32), jnp.int32),
        pltpu.VMEM((2 * C1, PACKW32), jnp.int32),
        pltpu.VMEM((C1, OW), jnp.float32),
        pltpu.VMEM((C1, OW), jnp.float32),
        pltpu.SemaphoreType.DMA,
        pltpu.SemaphoreType.DMA,
        pltpu.SemaphoreType.DMA,
        pltpu.SemaphoreType.DMA,
    ],
)
def _sc_edge(packAll, comb, out_e,
             idx0, idx1, gb0, gb1, ob0, ob1, gs0, gs1, ws0, ws1):
    wid = lax.axis_index("s") * NC + lax.axis_index("c")
    tb2 = wid * 2 * EPT1
    obase = wid * EPT1
    bufs = ((idx0, gb0, ob0, gs0, ws0), (idx1, gb1, ob1, gs1, ws1))

    for par in range(2):
        idx, gb, ob, gs, ws = bufs[par]
        pltpu.sync_copy(comb.at[pl.ds(tb2 + par * 2 * C1, 2 * C1)], idx)
        pltpu.async_copy(packAll.at[idx], gb, gs)

    def body(h, carry):
        for par in range(2):
            idx, gb, ob, gs, ws = bufs[par]
            g = 2 * h + par
            pltpu.make_async_copy(packAll.at[idx], gb, gs).wait()

            @pl.when(h > 0)
            def _():
                pltpu.make_async_copy(
                    ob, out_e.at[pl.ds(obase, C1)], ws).wait()

            def edge(i):
                acc = jnp.zeros((L,), jnp.float32)
                acr = jnp.zeros((L,), jnp.float32)
                acc_c = jnp.zeros((L,), jnp.float32)
                for j in range(D // (2 * L)):
                    ra, rb = _bf2(gb[i, pl.ds(L * j, L)])
                    ca, cb = _bf2(gb[C1 + i, pl.ds(L * j, L)])
                    acc = acc + ra * ca + rb * cb
                    acr = acr + ra * ra + rb * rb
                    acc_c = acc_c + ca * ca + cb * cb
                ob[i, pl.ds(256, L)] = acc
                ob[i, pl.ds(272, L)] = acr
                ob[i, pl.ds(288, L)] = acc_c
                for j in range(D // (2 * L)):
                    ra, rb = _bf2(gb[i, pl.ds(D // 2 + L * j, L)])
                    ca, cb = _bf2(gb[C1 + i, pl.ds(D // 2 + L * j, L)])
                    ob[i, pl.ds(2 * L * j, L)] = ra + ca
                    ob[i, pl.ds(2 * L * j + L, L)] = rb + cb

            plsc.parallel_loop(0, C1, unroll=2)(edge)
            pltpu.async_copy(ob, out_e.at[pl.ds(obase + g * C1, C1)], ws)

            @pl.when(g + 2 < NCH1)
            def _():
                pltpu.sync_copy(
                    comb.at[pl.ds(tb2 + (g + 2) * 2 * C1, 2 * C1)], idx)
                pltpu.async_copy(packAll.at[idx], gb, gs)
        return carry

    lax.fori_loop(0, NCH1 // 2, body, 0)
    for par in range(2):
        idx, gb, ob, gs, ws = bufs[par]
        pltpu.make_async_copy(ob, out_e.at[pl.ds(obase, C1)], ws).wait()


def _edge_body(e_ref, dd_ref, em_ref, w1d_ref, b1_ref, w2_ref, b2_ref,
               srep_ref):
    ev = e_ref[...]
    dot = jnp.sum(ev[:, 256:272], axis=1, keepdims=True)
    art_r = jnp.sqrt(jnp.maximum(jnp.sum(ev[:, 272:288], axis=1,
                                         keepdims=True), MIN_NORM))
    art_c = jnp.sqrt(jnp.maximum(jnp.sum(ev[:, 288:304], axis=1,
                                         keepdims=True), MIN_NORM))
    nr = jnp.tanh(art_r)
    nc = jnp.tanh(art_c)
    x2r = nr * nr
    y2 = nc * nc
    tr = art_r / nr
    tc_ = art_c / nc
    xy = dot / (tr * tc_)
    a = 1.0 - 2.0 * xy + y2
    b = 1.0 - x2r
    den = jnp.maximum(1.0 - 2.0 * xy + x2r * y2, MIN_NORM)
    nsq = (a * a * x2r - 2.0 * a * b * xy + b * b * y2) / (den * den)
    nn = jnp.sqrt(jnp.maximum(nsq, MIN_NORM))
    u = jnp.clip(nn, -1.0 + EPS, 1.0 - EPS)
    dist = jnp.log1p(u) - jnp.log1p(-u)
    z = (ev[:, 0:256] + dist * w1d_ref[0:1, :] + dd_ref[...] * w1d_ref[1:2, :]
         + b1_ref[...])
    h = z / (1.0 + jnp.exp(-z))
    s = jnp.dot(h, w2_ref[...], preferred_element_type=jnp.float32) + b2_ref[...]
    score = em_ref[...] / (1.0 + jnp.exp(-s))
    srep_ref[...] = jnp.broadcast_to(score, (score.shape[0], 16))


_edge_call = pl.pallas_call(
    _edge_body,
    grid=(E_PAD // BE,),
    in_specs=[
        pl.BlockSpec((BE, OW), lambda n: (n, 0)),
        pl.BlockSpec((BE, 1), lambda n: (n, 0)),
        pl.BlockSpec((BE, 1), lambda n: (n, 0)),
        pl.BlockSpec((2, D), lambda n: (0, 0)),
        pl.BlockSpec((1, D), lambda n: (0, 0)),
        pl.BlockSpec((D, 1), lambda n: (0, 0)),
        pl.BlockSpec((1, 1), lambda n: (0, 0)),
    ],
    out_specs=pl.BlockSpec((BE, 16), lambda n: (n, 0)),
    out_shape=jax.ShapeDtypeStruct((E_PAD, 16), jnp.float32),
)


@functools.partial(
    pl.kernel,
    out_type=[
        jax.ShapeDtypeStruct((N_PAD, 128), jnp.float32),
        jax.ShapeDtypeStruct((N_PAD, 128), jnp.float32),
    ],
    mesh=_mesh,
    scratch_types=[
        pltpu.VMEM((C2,), jnp.int32),
        pltpu.VMEM((C2,), jnp.int32),
        pltpu.VMEM((C2,), jnp.int32),
        pltpu.VMEM((C2,), jnp.int32),
        pltpu.VMEM((C2, 16), jnp.float32),
        pltpu.VMEM((C2, 16), jnp.float32),
        pltpu.VMEM((C2, 128), jnp.float32),
        pltpu.VMEM((C2, 128), jnp.float32),
        pltpu.VMEM((C2, 128), jnp.float32),
        pltpu.VMEM((C2, 128), jnp.float32),
        pltpu.VMEM_SHARED((N_PAD, 128), jnp.float32),
        pltpu.SemaphoreType.DMA,
        pltpu.SemaphoreType.DMA,
        pltpu.SemaphoreType.DMA,
        pltpu.SemaphoreType.DMA,
    ],
)
def _sc_scatter(xt2, rowi, coli, srep, zrows, aggL_out, aggR_out,
                ic0, ic1, ir0, ir1, sb0, sb1, gb0, gb1, vb0, vb1,
                acc, gs0, gs1, ss0, ss1):
    cid = lax.axis_index("c")
    sid = lax.axis_index("s")
    pltpu.sync_copy(zrows, acc.at[pl.ds(sid * NPT, NPT)])
    plsc.subcore_barrier()
    off = cid * N
    ebase = sid * EPT3
    bufs = ((ic0, ir0, sb0, gb0, vb0, gs0, ss0),
            (ic1, ir1, sb1, gb1, vb1, gs1, ss1))

    def load(par, g):
        ic, ir, sb, gb, vb, gs, ss = bufs[par]
        base = ebase + g * C2
        pltpu.sync_copy(coli.at[pl.ds(base, C2)], ic)
        pltpu.sync_copy(rowi.at[pl.ds(base, C2)], ir)
        pltpu.sync_copy(srep.at[pl.ds(base, C2)], sb)
        for q in range(C2 // L):
            ic[pl.ds(q * L, L)] = ic[pl.ds(q * L, L)] + off
        pltpu.async_copy(xt2.at[ic], gb, gs)

    for par in range(2):
        load(par, par)

    def body(h, carry):
        for par in range(2):
            ic, ir, sb, gb, vb, gs, ss = bufs[par]
            g = 2 * h + par
            pltpu.make_async_copy(xt2.at[ic], gb, gs).wait()

            @pl.when(h > 0)
            def _():
                pltpu.make_async_copy(vb, acc.at[ir], ss).wait()

            def edge(i):
                sv = sb[i, :]
                for j in range(128 // L):
                    vb[i, pl.ds(L * j, L)] = gb[i, pl.ds(L * j, L)] * sv

            plsc.parallel_loop(0, C2, unroll=2)(edge)
            pltpu.async_copy(vb, acc.at[ir], ss, add=True)

            @pl.when(g + 2 < NCH3)
            def _():
                load(par, g + 2)
        return carry

    lax.fori_loop(0, NCH3 // 2, body, 0)
    for par in range(2):
        ic, ir, sb, gb, vb, gs, ss = bufs[par]
        pltpu.make_async_copy(vb, acc.at[ir], ss).wait()
    plsc.subcore_barrier()

    @pl.when(cid == 0)
    def _():
        pltpu.sync_copy(acc.at[pl.ds(sid * NPT, NPT)],
                        aggL_out.at[pl.ds(sid * NPT, NPT)])

    @pl.when(cid == 1)
    def _():
        pltpu.sync_copy(acc.at[pl.ds(sid * NPT, NPT)],
                        aggR_out.at[pl.ds(sid * NPT, NPT)])


def _post_body(aL_ref, aR_ref, xt_ref, wm1_ref, bm1_ref, wm2_ref, bm2_ref,
               out_ref):
    agg = jnp.concatenate([aL_ref[...], aR_ref[...]], axis=1) * 0.01
    z = jnp.dot(agg, wm1_ref[...], preferred_element_type=jnp.float32) + bm1_ref[...]
    h = z / (1.0 + jnp.exp(-z))
    u = (jnp.dot(h, wm2_ref[...], preferred_element_type=jnp.float32)
         + bm2_ref[...] + xt_ref[...])
    nsq = jnp.sum(u * u, axis=1, keepdims=True)
    n = jnp.sqrt(jnp.maximum(nsq, MIN_NORM))
    out_ref[...] = jnp.tanh(n) * u / n


_post_call = pl.pallas_call(
    _post_body,
    grid=(N // BN,),
    in_specs=[
        pl.BlockSpec((BN, 128), lambda n: (n, 0)),
        pl.BlockSpec((BN, 128), lambda n: (n, 0)),
        pl.BlockSpec((BN, D), lambda n: (n, 0)),
        pl.BlockSpec((D, D), lambda n: (0, 0)),
        pl.BlockSpec((1, D), lambda n: (0, 0)),
        pl.BlockSpec((D, D), lambda n: (0, 0)),
        pl.BlockSpec((1, D), lambda n: (0, 0)),
    ],
    out_specs=pl.BlockSpec((BN, D), lambda n: (n, 0)),
    out_shape=jax.ShapeDtypeStruct((N, D), jnp.float32),
)


def kernel(x, distances, edges, node_mask, edge_mask,
           W_att1, b_att1, W_att2, b_att2, W_m1, b_m1, W_m2, b_m2):
    row = edges[0]
    col = edges[1]
    rowp = jnp.pad(row, (0, E_PAD - E))
    colp = jnp.pad(col, (0, E_PAD - E))
    comb = jnp.concatenate(
        [rowp.reshape(-1, C1), colp.reshape(-1, C1) + N], axis=1).reshape(-1)
    dd = jnp.pad(distances, ((0, E_PAD - E), (0, 0)))
    em = jnp.pad(edge_mask, ((0, E_PAD - E), (0, 0)))
    w1ab = jnp.stack([W_att1[:D], W_att1[D:2 * D]])
    w1d = W_att1[2 * D:]
    b1 = b_att1.reshape(1, D)
    b2 = b_att2.reshape(1, 1)
    bm1 = b_m1.reshape(1, D)
    bm2 = b_m2.reshape(1, D)
    zrows = jnp.zeros((NPT, 128), jnp.float32)

    packAll, xtL, xtR, x_tan = _prep_call(x, w1ab)
    packAll32 = lax.bitcast_convert_type(
        packAll.reshape(2 * N, PACKW32, 2), jnp.int32)
    xt2 = jnp.concatenate([xtL, xtR], axis=0)
    e_out = _sc_edge(packAll32, comb)
    srep = _edge_call(e_out, dd, em, w1d[:, _PERM_D], b1[:, _PERM_D],
                      W_att2[_PERM_D], b2)
    aggL, aggR = _sc_scatter(xt2, rowp, colp, srep, zrows)
    out = _post_call(aggL, aggR, x_tan, W_m1, bm1, W_m2, bm2)
    return out
